# baseline (speedup 1.0000x reference)
import functools

import jax
import jax.numpy as jnp
from jax import lax
from jax.experimental import pallas as pl
from jax.experimental.pallas import tpu as pltpu
from jax.experimental.pallas import tpu_sc as plsc

T = 2048
W = 192 * 14 * 14
NV = 64
L = 16
CW = 1280
NW = 32
B = 8
NBATCH = T // B
JW = CW // L
HARM_PAD = 2064


def _iota16():
    return lax.iota(jnp.int32, L)


def _splat_i32(v):
    return jnp.full((L,), v, dtype=jnp.int32)


def _body(x_hbm, vid_hbm, out_hbm,
          vid_v, harm_v, counts_v, starts_v, alpha_v, acc_v, buf_v,
          sem0, sem1):
    c = lax.axis_index("c")
    s = lax.axis_index("s")
    wid = c * 16 + s
    woff = ((wid * 147) >> 4) * 128

    iota = _iota16()
    zero_f = jnp.zeros((L,), jnp.float32)
    one_i = jnp.ones((L,), jnp.int32)

    pltpu.sync_copy(vid_hbm, vid_v)

    def harm_step(i, carry):
        base = i * L
        inv = 1.0 / (base + iota + 1).astype(jnp.float32)
        tot = carry + plsc.cumsum(inv)
        harm_v[pl.ds(base, L)] = tot
        return plsc.load_gather(harm_v, [_splat_i32(base + L - 1)])

    lax.fori_loop(0, T // L, harm_step, zero_f)

    for k in range(NV // L):
        counts_v[pl.ds(k * L, L)] = jnp.zeros((L,), jnp.int32)

    def count_step(i, _):
        chunk = vid_v[pl.ds(i * L, L)]
        plsc.addupdate_scatter(counts_v, [chunk], one_i)
        return 0

    lax.fori_loop(0, T // L, count_step, 0)

    carry_i = jnp.zeros((L,), jnp.int32)
    for k in range(NV // L):
        cnt = counts_v[pl.ds(k * L, L)]
        incl = plsc.cumsum(cnt)
        starts_v[pl.ds(k * L, L)] = carry_i + incl - cnt
        last = _splat_i32(k * L + (L - 1))
        carry_i = (plsc.load_gather(starts_v, [last])
                   + plsc.load_gather(counts_v, [last]))

    def alpha_step(i, _):
        base = i * L
        vid = vid_v[pl.ds(base, L)]
        st = plsc.load_gather(starts_v, [vid])
        n = plsc.load_gather(counts_v, [vid])
        t1 = base + iota + 1 - st
        hn = plsc.load_gather(harm_v, [n - 1])
        ht_raw = plsc.load_gather(harm_v, [jnp.maximum(t1 - 2, 0)])
        ht = jnp.where(t1 == 1, jnp.float32(0.0), ht_raw)
        nf = n.astype(jnp.float32)
        a = 2.0 * (nf - t1.astype(jnp.float32) + 1.0) - (nf + 1.0) * (hn - ht)
        a = jnp.where(n == 1, jnp.float32(1.0), a)
        alpha_v[pl.ds(base, L)] = a
        return 0

    lax.fori_loop(0, T // L, alpha_step, 0)

    def zero_step(r, _):
        for j in range(JW):
            acc_v[r, pl.ds(j * L, L)] = zero_f
        return 0

    lax.fori_loop(0, NV, zero_step, 0)

    def start_batch(batch, bi):
        return pltpu.async_copy(
            x_hbm.at[pl.ds(batch * B, B), pl.ds(woff, CW)],
            buf_v.at[bi], sem0 if bi == 0 else sem1)

    def wait_batch(batch, bi):
        pltpu.make_async_copy(
            x_hbm.at[pl.ds(batch * B, B), pl.ds(woff, CW)],
            buf_v.at[bi], sem0 if bi == 0 else sem1).wait()

    start_batch(0, 0)
    start_batch(1, 1)

    def frame_step(f, batch, bi):
        t = batch * B + f
        tv = _splat_i32(t)
        vid = plsc.load_gather(vid_v, [tv])
        av = plsc.load_gather(alpha_v, [tv])
        for j in range(JW):
            xv = buf_v[bi, f, pl.ds(j * L, L)]
            plsc.addupdate_scatter(acc_v, [vid, iota + (j * L)], xv * av)

    def pair_step(g, _):
        for bi in range(2):
            batch = 2 * g + bi
            wait_batch(batch, bi)
            lax.fori_loop(0, B, lambda f, _2: (frame_step(f, batch, bi), 0)[1], 0)

            @pl.when(g < NBATCH // 2 - 1)
            def _():
                start_batch(batch + 2, bi)
        return 0

    lax.fori_loop(0, NBATCH // 2, pair_step, 0)

    pltpu.sync_copy(acc_v, out_hbm.at[:, pl.ds(woff, CW)])


@jax.jit
def _rank_pool(x2d, vidids):
    mesh = plsc.VectorSubcoreMesh(core_axis_name="c", subcore_axis_name="s")
    return pl.kernel(
        _body,
        out_type=jax.ShapeDtypeStruct((NV, W), jnp.float32),
        mesh=mesh,
        compiler_params=pltpu.CompilerParams(needs_layout_passes=False),
        scratch_types=[
            pltpu.VMEM((T,), jnp.int32),
            pltpu.VMEM((T,), jnp.float32),
            pltpu.VMEM((NV,), jnp.int32),
            pltpu.VMEM((NV,), jnp.int32),
            pltpu.VMEM((T,), jnp.float32),
            pltpu.VMEM((NV, CW), jnp.float32),
            pltpu.VMEM((2, B, CW), jnp.float32),
            pltpu.SemaphoreType.DMA,
            pltpu.SemaphoreType.DMA,
        ],
    )(x2d, vidids)


def kernel(x, vidids):
    out = _rank_pool(x.reshape(T, W), vidids)
    return out.reshape(NV, 192, 14, 14)


# speedup vs baseline: 1.2035x; 1.2035x over previous
import functools

import jax
import jax.numpy as jnp
from jax import lax
from jax.experimental import pallas as pl
from jax.experimental.pallas import tpu as pltpu
from jax.experimental.pallas import tpu_sc as plsc

T = 2048
W = 192 * 14 * 14
NV = 64
L = 16
CW = 1184
HW = CW // 2
JH = HW // L
NW = 32
B = 8
NBATCH = T // B


def _iota16():
    return lax.iota(jnp.int32, L)


def _splat_i32(v):
    return jnp.full((L,), v, dtype=jnp.int32)


def _body(x_hbm, vid_hbm, out_hbm,
          vid_v, harm_v, counts_v, starts_v, alpha_v, stg_v, buf_v, cnt_s,
          sem0, sem1):
    c = lax.axis_index("c")
    s = lax.axis_index("s")
    wid = c * 16 + s
    woff = jnp.minimum(wid * CW, W - CW)

    iota = _iota16()
    zero_f = jnp.zeros((L,), jnp.float32)
    one_i = jnp.ones((L,), jnp.int32)

    pltpu.sync_copy(vid_hbm, vid_v.at[pl.ds(0, T)])

    def harm_step(i, carry):
        base = i * L
        inv = 1.0 / (base + iota + 1).astype(jnp.float32)
        tot = carry + plsc.cumsum(inv)
        harm_v[pl.ds(base, L)] = tot
        return plsc.load_gather(harm_v, [_splat_i32(base + L - 1)])

    lax.fori_loop(0, T // L, harm_step, zero_f)

    for k in range(NV // L):
        counts_v[pl.ds(k * L, L)] = jnp.zeros((L,), jnp.int32)

    def count_step(i, _):
        chunk = vid_v[pl.ds(i * L, L)]
        plsc.addupdate_scatter(counts_v, [chunk], one_i)
        return 0

    lax.fori_loop(0, T // L, count_step, 0)

    carry_i = jnp.zeros((L,), jnp.int32)
    for k in range(NV // L):
        cnt = counts_v[pl.ds(k * L, L)]
        for l in range(L):
            cnt_s[k * L + l] = cnt[l]
        incl = plsc.cumsum(cnt)
        starts_v[pl.ds(k * L, L)] = carry_i + incl - cnt
        last = _splat_i32(k * L + (L - 1))
        carry_i = (plsc.load_gather(starts_v, [last])
                   + plsc.load_gather(counts_v, [last]))

    def alpha_step(i, _):
        base = i * L
        vid = vid_v[pl.ds(base, L)]
        st = plsc.load_gather(starts_v, [vid])
        n = plsc.load_gather(counts_v, [vid])
        t1 = base + iota + 1 - st
        hn = plsc.load_gather(harm_v, [n - 1])
        ht_raw = plsc.load_gather(harm_v, [jnp.maximum(t1 - 2, 0)])
        ht = jnp.where(t1 == 1, jnp.float32(0.0), ht_raw)
        nf = n.astype(jnp.float32)
        a = 2.0 * (nf - t1.astype(jnp.float32) + 1.0) - (nf + 1.0) * (hn - ht)
        a = jnp.where(n == 1, jnp.float32(1.0), a)
        alpha_v[pl.ds(base, L)] = a
        return 0

    lax.fori_loop(0, T // L, alpha_step, 0)

    vid0 = vid_v[pl.ds(0, L)][0]

    def half_step(h, _h):
        colbase = woff + h * HW

        def start_batch(batch, bi):
            return pltpu.async_copy(
                x_hbm.at[pl.ds(batch * B, B), pl.ds(colbase, HW)],
                buf_v.at[bi], sem0 if bi == 0 else sem1)

        def wait_batch(batch, bi):
            pltpu.make_async_copy(
                x_hbm.at[pl.ds(batch * B, B), pl.ds(colbase, HW)],
                buf_v.at[bi], sem0 if bi == 0 else sem1).wait()

        start_batch(0, 0)
        start_batch(1, 1)

        def pair_step(g, carry):
            for bi in range(2):
                batch = 2 * g + bi
                wait_batch(batch, bi)
                vv = vid_v[pl.ds(batch * B, L)]
                av = alpha_v[pl.ds(batch * B, L)]
                for f in range(B):
                    prev_vid, acc = carry[0], carry[1:]
                    vid = vv[f]

                    @pl.when(vid != prev_vid)
                    def _():
                        for j in range(JH):
                            stg_v[prev_vid, pl.ds(j * L, L)] = acc[j]

                    afv = jnp.full((L,), av[f])
                    for jg in range(0, JH, 8):
                        js = range(jg, min(jg + 8, JH))
                        xs = [buf_v[bi, f, pl.ds(j * L, L)] for j in js]
                        ws = [x * afv for x in xs]
                        acc = (list(acc[:jg])
                               + [a + w for a, w in zip(acc[jg:], ws)]
                               + list(acc[jg + len(ws):]))
                    carry = (vid,) + tuple(acc)

                @pl.when(g < NBATCH // 2 - 1)
                def _():
                    start_batch(batch + 2, bi)
            return carry

        init = (vid0,) + tuple(zero_f for _ in range(JH))
        final = lax.fori_loop(0, NBATCH // 2, pair_step, init)
        last_vid, last_acc = final[0], final[1:]
        for j in range(JH):
            stg_v[last_vid, pl.ds(j * L, L)] = last_acc[j]

        def diff_step(v, last):
            n = cnt_s[v]
            nz = n > 0
            new_last = []
            for j in range(JH):
                row = stg_v[v, pl.ds(j * L, L)]
                stg_v[v, pl.ds(j * L, L)] = jnp.where(nz, row - last[j], 0.0)
                new_last.append(jnp.where(nz, row, last[j]))
            return tuple(new_last)

        lax.fori_loop(0, NV, diff_step, tuple(zero_f for _ in range(JH)))

        pltpu.sync_copy(stg_v, out_hbm.at[:, pl.ds(colbase, HW)])
        return _h

    lax.fori_loop(0, 2, half_step, 0)


@jax.jit
def _rank_pool(x2d, vidids):
    mesh = plsc.VectorSubcoreMesh(core_axis_name="c", subcore_axis_name="s")
    return pl.kernel(
        _body,
        out_type=jax.ShapeDtypeStruct((NV, W), jnp.float32),
        mesh=mesh,
        compiler_params=pltpu.CompilerParams(
            needs_layout_passes=False, use_tc_tiling_on_sc=False),
        scratch_types=[
            pltpu.VMEM((T + L,), jnp.int32),
            pltpu.VMEM((T,), jnp.float32),
            pltpu.VMEM((NV,), jnp.int32),
            pltpu.VMEM((NV,), jnp.int32),
            pltpu.VMEM((T + L,), jnp.float32),
            pltpu.VMEM((NV, HW), jnp.float32),
            pltpu.VMEM((2, B, HW), jnp.float32),
            pltpu.SMEM((NV,), jnp.int32),
            pltpu.SemaphoreType.DMA,
            pltpu.SemaphoreType.DMA,
        ],
    )(x2d, vidids)


def kernel(x, vidids):
    out = _rank_pool(x.reshape(T, W), vidids)
    return out.reshape(NV, 192, 14, 14)
